# baseline (speedup 1.0000x reference)
import functools

import jax
import jax.numpy as jnp
from jax import lax
from jax.experimental import pallas as pl
from jax.experimental.pallas import tpu as pltpu
from jax.experimental.pallas import tpu_sc as plsc

_BLK = 4096
_NC = 2
_NS = 16
_LANES = 16


def _tc_prefix_body(x_ref, w1_ref, b1t_ref, w2t_ref, b2_ref, n_ref,
                    p_ref, carry_ref):
    i = pl.program_id(0)

    @pl.when(i == 0)
    def _():
        carry_ref[0, 0] = 0.0

    x = x_ref[...]
    ht = lax.dot_general(w1_ref[...], x,
                         (((0,), (1,)), ((), ())),
                         preferred_element_type=jnp.float32)
    ht = ht + b1t_ref[...]
    u = ht * 0.5
    ht = u + u * jnp.tanh(u)
    yit = jnp.dot(w2t_ref[...].astype(jnp.bfloat16), ht.astype(jnp.bfloat16),
                  preferred_element_type=jnp.float32)
    yit = yit + b2_ref[0, 0]
    cols = lax.broadcasted_iota(jnp.int32, (1, _BLK), 1) + i * _BLK
    yit = jnp.where(cols < n_ref[0, 0], yit, 0.0)
    zro = jnp.zeros((1, _BLK), jnp.float32)
    s = yit
    k = 1
    while k < _BLK:
        s = s + jnp.concatenate([zro[:, :k], s[:, : _BLK - k]], axis=1)
        k *= 2
    carry = carry_ref[0, 0]
    p_ref[...] = (s - yit + carry).reshape(1, 1, _BLK)
    carry_ref[0, 0] = carry + jnp.sum(yit)


def _sc_csr_diff(n_mol, spw):
    mesh = plsc.VectorSubcoreMesh(core_axis_name="c", subcore_axis_name="s")
    chunk = spw + _LANES

    @functools.partial(
        pl.kernel,
        mesh=mesh,
        out_type=jax.ShapeDtypeStruct((n_mol,), jnp.float32),
        scratch_types=[
            pltpu.VMEM((chunk,), jnp.int32),
            pltpu.VMEM((chunk,), jnp.float32),
            pltpu.VMEM((spw,), jnp.float32),
            pltpu.SemaphoreType.DMA,
        ],
    )
    def run(seg_hbm, p_hbm, out_hbm, idx_v, vals_v, out_v, sem):
        w = lax.axis_index("s") * _NC + lax.axis_index("c")
        base = jnp.minimum(w * spw, n_mol - spw)
        zeros16 = jnp.zeros((_LANES,), jnp.int32)
        for k in range(chunk // _LANES):
            idx_v[pl.ds(k * _LANES, _LANES)] = zeros16
        pltpu.sync_copy(seg_hbm.at[pl.ds(base, spw + 1)],
                        idx_v.at[pl.ds(0, spw + 1)])
        pltpu.async_copy(p_hbm.at[idx_v], vals_v, sem).wait()
        for k in range(spw // _LANES):
            a = vals_v[pl.ds(k * _LANES, _LANES)]
            b = vals_v[pl.ds(k * _LANES + 1, _LANES)]
            out_v[pl.ds(k * _LANES, _LANES)] = b - a
        pltpu.sync_copy(out_v, out_hbm.at[pl.ds(base, spw)])

    return run


def kernel(scalar_representation, atomic_numbers, seg_m, W1, b1, W2, b2):
    del atomic_numbers
    n, d_in = scalar_representation.shape
    d_hid = W1.shape[1]
    n_mol = seg_m.shape[0] - 1
    nblk = (n + _BLK - 1) // _BLK
    npad = nblk * _BLK

    n_arr = jnp.full((1, 1), n, dtype=jnp.int32)

    p = pl.pallas_call(
        _tc_prefix_body,
        grid=(nblk,),
        in_specs=[
            pl.BlockSpec((_BLK, d_in), lambda i: (i, 0)),
            pl.BlockSpec((d_in, d_hid), lambda i: (0, 0)),
            pl.BlockSpec((d_hid, 1), lambda i: (0, 0)),
            pl.BlockSpec((1, d_hid), lambda i: (0, 0)),
            pl.BlockSpec((1, 1), lambda i: (0, 0), memory_space=pltpu.SMEM),
            pl.BlockSpec((1, 1), lambda i: (0, 0), memory_space=pltpu.SMEM),
        ],
        out_specs=pl.BlockSpec((1, 1, _BLK), lambda i: (i, 0, 0)),
        out_shape=jax.ShapeDtypeStruct((nblk, 1, _BLK), jnp.float32),
        scratch_shapes=[pltpu.SMEM((1, 1), jnp.float32)],
    )(scalar_representation, W1, b1.reshape(d_hid, 1),
      W2.reshape(1, d_hid), b2.reshape(1, 1), n_arr)

    n_w = _NC * _NS
    spw = (-(-n_mol // n_w) + _LANES - 1) // _LANES * _LANES
    return _sc_csr_diff(n_mol, spw)(seg_m.astype(jnp.int32), p.reshape(npad))


# speedup vs baseline: 1.1406x; 1.1406x over previous
import functools

import jax
import jax.numpy as jnp
from jax import lax
from jax.experimental import pallas as pl
from jax.experimental.pallas import tpu as pltpu
from jax.experimental.pallas import tpu_sc as plsc

_BLK = 8192
_NC = 2
_NS = 16
_LANES = 16


def _tc_prefix_body(x_ref, w1_ref, b1t_ref, w2t_ref, b2_ref, n_ref,
                    p_ref, carry_ref):
    i = pl.program_id(0)

    @pl.when(i == 0)
    def _():
        carry_ref[0, 0] = 0.0

    x = x_ref[...]
    ht = lax.dot_general(w1_ref[...], x,
                         (((0,), (1,)), ((), ())),
                         preferred_element_type=jnp.float32)
    ht = ht + b1t_ref[...]
    u = ht * 0.5
    ht = u + u * jnp.tanh(u)
    yit = jnp.dot(w2t_ref[...].astype(jnp.bfloat16), ht.astype(jnp.bfloat16),
                  preferred_element_type=jnp.float32)
    yit = yit + b2_ref[0, 0]
    cols = lax.broadcasted_iota(jnp.int32, (1, _BLK), 1) + i * _BLK
    yit = jnp.where(cols < n_ref[0, 0], yit, 0.0)
    zro = jnp.zeros((1, _BLK), jnp.float32)
    s = yit
    k = 1
    while k < _BLK:
        s = s + jnp.concatenate([zro[:, :k], s[:, : _BLK - k]], axis=1)
        k *= 2
    carry = carry_ref[0, 0]
    p_ref[...] = (s - yit + carry).reshape(1, 1, _BLK)
    carry_ref[0, 0] = carry + jnp.sum(yit)


def _sc_csr_diff(n_mol, spw):
    mesh = plsc.VectorSubcoreMesh(core_axis_name="c", subcore_axis_name="s")
    chunk = spw + _LANES

    @functools.partial(
        pl.kernel,
        mesh=mesh,
        out_type=jax.ShapeDtypeStruct((n_mol,), jnp.float32),
        scratch_types=[
            pltpu.VMEM((chunk,), jnp.int32),
            pltpu.VMEM((chunk,), jnp.float32),
            pltpu.VMEM((spw,), jnp.float32),
            pltpu.SemaphoreType.DMA,
        ],
    )
    def run(seg_hbm, p_hbm, out_hbm, idx_v, vals_v, out_v, sem):
        w = lax.axis_index("s") * _NC + lax.axis_index("c")
        base = jnp.minimum(w * spw, n_mol - spw)
        zeros16 = jnp.zeros((_LANES,), jnp.int32)
        for k in range(chunk // _LANES):
            idx_v[pl.ds(k * _LANES, _LANES)] = zeros16
        pltpu.sync_copy(seg_hbm.at[pl.ds(base, spw + 1)],
                        idx_v.at[pl.ds(0, spw + 1)])
        pltpu.async_copy(p_hbm.at[idx_v], vals_v, sem).wait()
        for k in range(spw // _LANES):
            a = vals_v[pl.ds(k * _LANES, _LANES)]
            b = vals_v[pl.ds(k * _LANES + 1, _LANES)]
            out_v[pl.ds(k * _LANES, _LANES)] = b - a
        pltpu.sync_copy(out_v, out_hbm.at[pl.ds(base, spw)])

    return run


def kernel(scalar_representation, atomic_numbers, seg_m, W1, b1, W2, b2):
    del atomic_numbers
    n, d_in = scalar_representation.shape
    d_hid = W1.shape[1]
    n_mol = seg_m.shape[0] - 1
    nblk = (n + _BLK - 1) // _BLK
    npad = nblk * _BLK

    n_arr = jnp.full((1, 1), n, dtype=jnp.int32)

    p = pl.pallas_call(
        _tc_prefix_body,
        grid=(nblk,),
        in_specs=[
            pl.BlockSpec((_BLK, d_in), lambda i: (i, 0)),
            pl.BlockSpec((d_in, d_hid), lambda i: (0, 0)),
            pl.BlockSpec((d_hid, 1), lambda i: (0, 0)),
            pl.BlockSpec((1, d_hid), lambda i: (0, 0)),
            pl.BlockSpec((1, 1), lambda i: (0, 0), memory_space=pltpu.SMEM),
            pl.BlockSpec((1, 1), lambda i: (0, 0), memory_space=pltpu.SMEM),
        ],
        out_specs=pl.BlockSpec((1, 1, _BLK), lambda i: (i, 0, 0)),
        out_shape=jax.ShapeDtypeStruct((nblk, 1, _BLK), jnp.float32),
        scratch_shapes=[pltpu.SMEM((1, 1), jnp.float32)],
    )(scalar_representation, W1, b1.reshape(d_hid, 1),
      W2.reshape(1, d_hid), b2.reshape(1, 1), n_arr)

    n_w = _NC * _NS
    spw = (-(-n_mol // n_w) + _LANES - 1) // _LANES * _LANES
    return _sc_csr_diff(n_mol, spw)(seg_m.astype(jnp.int32), p.reshape(npad))


# speedup vs baseline: 1.1856x; 1.0394x over previous
import functools

import jax
import jax.numpy as jnp
from jax import lax
from jax.experimental import pallas as pl
from jax.experimental.pallas import tpu as pltpu
from jax.experimental.pallas import tpu_sc as plsc

_BLK = 8192
_NC = 2
_NS = 16
_LANES = 16


def _tc_prefix_body(x_ref, w1_ref, b1t_ref, w2t_ref, b2_ref, n_ref,
                    p_ref, carry_ref):
    i = pl.program_id(0)

    @pl.when(i == 0)
    def _():
        carry_ref[0, 0] = 0.0

    x = x_ref[...]
    ht = lax.dot_general(w1_ref[...], x,
                         (((0,), (1,)), ((), ())),
                         preferred_element_type=jnp.float32)
    ht = ht + b1t_ref[...]
    u = ht * 0.5
    ht = u + u * jnp.tanh(u)
    yit = jnp.dot(w2t_ref[...].astype(jnp.bfloat16), ht.astype(jnp.bfloat16),
                  preferred_element_type=jnp.float32)
    yit = yit + b2_ref[0, 0]
    cols = lax.broadcasted_iota(jnp.int32, (1, _BLK), 1) + i * _BLK
    yit = jnp.where(cols < n_ref[0, 0], yit, 0.0)
    zro = jnp.zeros((1, _BLK), jnp.float32)
    s = yit
    k = 1
    while k < _BLK:
        s = s + jnp.concatenate([zro[:, :k], s[:, : _BLK - k]], axis=1)
        k *= 2
    carry = carry_ref[0, 0]
    p_ref[...] = (s - yit + carry).reshape(1, 1, _BLK)
    carry_ref[0, 0] = carry + jnp.sum(yit)


def _sc_csr_diff(n_mol, spw):
    mesh = plsc.VectorSubcoreMesh(core_axis_name="c", subcore_axis_name="s",
                                  num_cores=1)
    chunk = spw + _LANES

    @functools.partial(
        pl.kernel,
        mesh=mesh,
        out_type=jax.ShapeDtypeStruct((n_mol,), jnp.float32),
        scratch_types=[
            pltpu.VMEM((chunk,), jnp.int32),
            pltpu.VMEM((chunk,), jnp.float32),
            pltpu.VMEM((spw,), jnp.float32),
            pltpu.SemaphoreType.DMA,
        ],
    )
    def run(seg_hbm, p_hbm, out_hbm, idx_v, vals_v, out_v, sem):
        w = lax.axis_index("s")
        base = jnp.minimum(w * spw, n_mol - spw)
        zeros16 = jnp.zeros((_LANES,), jnp.int32)
        for k in range(chunk // _LANES):
            idx_v[pl.ds(k * _LANES, _LANES)] = zeros16
        pltpu.sync_copy(seg_hbm.at[pl.ds(base, spw + 1)],
                        idx_v.at[pl.ds(0, spw + 1)])
        pltpu.async_copy(p_hbm.at[idx_v], vals_v, sem).wait()
        for k in range(spw // _LANES):
            a = vals_v[pl.ds(k * _LANES, _LANES)]
            b = vals_v[pl.ds(k * _LANES + 1, _LANES)]
            out_v[pl.ds(k * _LANES, _LANES)] = b - a
        pltpu.sync_copy(out_v, out_hbm.at[pl.ds(base, spw)])

    return run


def kernel(scalar_representation, atomic_numbers, seg_m, W1, b1, W2, b2):
    del atomic_numbers
    n, d_in = scalar_representation.shape
    d_hid = W1.shape[1]
    n_mol = seg_m.shape[0] - 1
    nblk = (n + _BLK - 1) // _BLK
    npad = nblk * _BLK

    n_arr = jnp.full((1, 1), n, dtype=jnp.int32)

    p = pl.pallas_call(
        _tc_prefix_body,
        grid=(nblk,),
        in_specs=[
            pl.BlockSpec((_BLK, d_in), lambda i: (i, 0)),
            pl.BlockSpec((d_in, d_hid), lambda i: (0, 0)),
            pl.BlockSpec((d_hid, 1), lambda i: (0, 0)),
            pl.BlockSpec((1, d_hid), lambda i: (0, 0)),
            pl.BlockSpec((1, 1), lambda i: (0, 0), memory_space=pltpu.SMEM),
            pl.BlockSpec((1, 1), lambda i: (0, 0), memory_space=pltpu.SMEM),
        ],
        out_specs=pl.BlockSpec((1, 1, _BLK), lambda i: (i, 0, 0)),
        out_shape=jax.ShapeDtypeStruct((nblk, 1, _BLK), jnp.float32),
        scratch_shapes=[pltpu.SMEM((1, 1), jnp.float32)],
    )(scalar_representation, W1, b1.reshape(d_hid, 1),
      W2.reshape(1, d_hid), b2.reshape(1, 1), n_arr)

    n_w = _NS
    spw = (-(-n_mol // n_w) + _LANES - 1) // _LANES * _LANES
    return _sc_csr_diff(n_mol, spw)(seg_m.astype(jnp.int32), p.reshape(npad))


# speedup vs baseline: 1.1939x; 1.0071x over previous
import functools

import jax
import jax.numpy as jnp
from jax import lax
from jax.experimental import pallas as pl
from jax.experimental.pallas import tpu as pltpu
from jax.experimental.pallas import tpu_sc as plsc

_BLK = 8192
_NC = 2
_NS = 16
_LANES = 16


def _tc_prefix_body(x_ref, w1_ref, b1t_ref, w2t_ref, b2_ref, n_ref,
                    p_ref, carry_ref):
    i = pl.program_id(0)

    @pl.when(i == 0)
    def _():
        carry_ref[0, 0] = 0.0

    x = x_ref[...]
    ht = lax.dot_general(w1_ref[...], x,
                         (((0,), (1,)), ((), ())),
                         preferred_element_type=jnp.float32)
    ht = ht + b1t_ref[...]
    u = ht * 0.5
    ht = u + u * jnp.tanh(u)
    yit = jnp.dot(w2t_ref[...].astype(jnp.bfloat16), ht.astype(jnp.bfloat16),
                  preferred_element_type=jnp.float32)
    yit = yit + b2_ref[0, 0]
    cols = lax.broadcasted_iota(jnp.int32, (1, _BLK), 1) + i * _BLK
    yit = jnp.where(cols < n_ref[0, 0], yit, 0.0)
    zro = jnp.zeros((1, _BLK), jnp.float32)
    s = yit
    k = 1
    while k < _BLK:
        s = s + jnp.concatenate([zro[:, :k], s[:, : _BLK - k]], axis=1)
        k *= 2
    carry = carry_ref[0, 0]
    p_ref[...] = (s - yit + carry).reshape(1, 1, _BLK)
    carry_ref[0, 0] = carry + jnp.sum(yit)


def _sc_csr_diff(n_mol, spw):
    mesh = plsc.VectorSubcoreMesh(core_axis_name="c", subcore_axis_name="s",
                                  num_cores=1, num_subcores=8)
    chunk = spw + _LANES

    @functools.partial(
        pl.kernel,
        mesh=mesh,
        out_type=jax.ShapeDtypeStruct((n_mol,), jnp.float32),
        scratch_types=[
            pltpu.VMEM((chunk,), jnp.int32),
            pltpu.VMEM((chunk,), jnp.float32),
            pltpu.VMEM((spw,), jnp.float32),
            pltpu.SemaphoreType.DMA,
        ],
    )
    def run(seg_hbm, p_hbm, out_hbm, idx_v, vals_v, out_v, sem):
        w = lax.axis_index("s")
        base = jnp.minimum(w * spw, n_mol - spw)
        zeros16 = jnp.zeros((_LANES,), jnp.int32)
        for k in range(chunk // _LANES):
            idx_v[pl.ds(k * _LANES, _LANES)] = zeros16
        pltpu.sync_copy(seg_hbm.at[pl.ds(base, spw + 1)],
                        idx_v.at[pl.ds(0, spw + 1)])
        pltpu.async_copy(p_hbm.at[idx_v], vals_v, sem).wait()
        for k in range(spw // _LANES):
            a = vals_v[pl.ds(k * _LANES, _LANES)]
            b = vals_v[pl.ds(k * _LANES + 1, _LANES)]
            out_v[pl.ds(k * _LANES, _LANES)] = b - a
        pltpu.sync_copy(out_v, out_hbm.at[pl.ds(base, spw)])

    return run


def kernel(scalar_representation, atomic_numbers, seg_m, W1, b1, W2, b2):
    del atomic_numbers
    n, d_in = scalar_representation.shape
    d_hid = W1.shape[1]
    n_mol = seg_m.shape[0] - 1
    nblk = (n + _BLK - 1) // _BLK
    npad = nblk * _BLK

    n_arr = jnp.full((1, 1), n, dtype=jnp.int32)

    p = pl.pallas_call(
        _tc_prefix_body,
        grid=(nblk,),
        in_specs=[
            pl.BlockSpec((_BLK, d_in), lambda i: (i, 0)),
            pl.BlockSpec((d_in, d_hid), lambda i: (0, 0)),
            pl.BlockSpec((d_hid, 1), lambda i: (0, 0)),
            pl.BlockSpec((1, d_hid), lambda i: (0, 0)),
            pl.BlockSpec((1, 1), lambda i: (0, 0), memory_space=pltpu.SMEM),
            pl.BlockSpec((1, 1), lambda i: (0, 0), memory_space=pltpu.SMEM),
        ],
        out_specs=pl.BlockSpec((1, 1, _BLK), lambda i: (i, 0, 0)),
        out_shape=jax.ShapeDtypeStruct((nblk, 1, _BLK), jnp.float32),
        scratch_shapes=[pltpu.SMEM((1, 1), jnp.float32)],
    )(scalar_representation, W1, b1.reshape(d_hid, 1),
      W2.reshape(1, d_hid), b2.reshape(1, 1), n_arr)

    n_w = 8
    spw = (-(-n_mol // n_w) + _LANES - 1) // _LANES * _LANES
    return _sc_csr_diff(n_mol, spw)(seg_m.astype(jnp.int32), p.reshape(npad))


# speedup vs baseline: 1.1953x; 1.0011x over previous
import functools

import jax
import jax.numpy as jnp
from jax import lax
from jax.experimental import pallas as pl
from jax.experimental.pallas import tpu as pltpu
from jax.experimental.pallas import tpu_sc as plsc

_BLK = 8192
_SC_WORKERS = 8
_LANES = 16


def _tc_prefix_body(x_ref, w1_ref, b1t_ref, w2t_ref, b2_ref, n_ref,
                    p_ref, carry_ref):
    i = pl.program_id(0)

    @pl.when(i == 0)
    def _():
        carry_ref[0, 0] = 0.0

    x = x_ref[...]
    ht = lax.dot_general(w1_ref[...], x,
                         (((0,), (1,)), ((), ())),
                         preferred_element_type=jnp.float32)
    ht = ht + b1t_ref[...]
    u = ht * 0.5
    ht = u + u * jnp.tanh(u)
    yit = jnp.dot(w2t_ref[...].astype(jnp.bfloat16), ht.astype(jnp.bfloat16),
                  preferred_element_type=jnp.float32)
    yit = yit + b2_ref[0, 0]
    cols = lax.broadcasted_iota(jnp.int32, (1, _BLK), 1) + i * _BLK
    yit = jnp.where(cols < n_ref[0, 0], yit, 0.0)
    zro = jnp.zeros((1, _BLK), jnp.float32)
    s = yit
    k = 1
    while k < _BLK:
        s = s + jnp.concatenate([zro[:, :k], s[:, : _BLK - k]], axis=1)
        k *= 2
    carry = carry_ref[0, 0]
    p_ref[...] = (s - yit + carry).reshape(1, 1, _BLK)
    carry_ref[0, 0] = carry + jnp.sum(yit)


def _sc_csr_diff(n_mol, spw):
    mesh = plsc.VectorSubcoreMesh(core_axis_name="c", subcore_axis_name="s",
                                  num_cores=1, num_subcores=_SC_WORKERS)
    chunk = spw + _LANES

    @functools.partial(
        pl.kernel,
        mesh=mesh,
        out_type=jax.ShapeDtypeStruct((n_mol,), jnp.float32),
        scratch_types=[
            pltpu.VMEM((chunk,), jnp.int32),
            pltpu.VMEM((chunk,), jnp.float32),
            pltpu.VMEM((spw,), jnp.float32),
            pltpu.SemaphoreType.DMA,
        ],
    )
    def run(seg_hbm, p_hbm, out_hbm, idx_v, vals_v, out_v, sem):
        w = lax.axis_index("s")
        base = jnp.minimum(w * spw, n_mol - spw)
        zeros16 = jnp.zeros((_LANES,), jnp.int32)
        for k in range(chunk // _LANES):
            idx_v[pl.ds(k * _LANES, _LANES)] = zeros16
        pltpu.sync_copy(seg_hbm.at[pl.ds(base, spw + 1)],
                        idx_v.at[pl.ds(0, spw + 1)])
        pltpu.async_copy(p_hbm.at[idx_v], vals_v, sem).wait()
        for k in range(spw // _LANES):
            a = vals_v[pl.ds(k * _LANES, _LANES)]
            b = vals_v[pl.ds(k * _LANES + 1, _LANES)]
            out_v[pl.ds(k * _LANES, _LANES)] = b - a
        pltpu.sync_copy(out_v, out_hbm.at[pl.ds(base, spw)])

    return run


def kernel(scalar_representation, atomic_numbers, seg_m, W1, b1, W2, b2):
    del atomic_numbers
    n, d_in = scalar_representation.shape
    d_hid = W1.shape[1]
    n_mol = seg_m.shape[0] - 1
    nblk = (n + _BLK - 1) // _BLK
    npad = nblk * _BLK

    n_arr = jnp.full((1, 1), n, dtype=jnp.int32)

    p = pl.pallas_call(
        _tc_prefix_body,
        grid=(nblk,),
        in_specs=[
            pl.BlockSpec((_BLK, d_in), lambda i: (i, 0)),
            pl.BlockSpec((d_in, d_hid), lambda i: (0, 0)),
            pl.BlockSpec((d_hid, 1), lambda i: (0, 0)),
            pl.BlockSpec((1, d_hid), lambda i: (0, 0)),
            pl.BlockSpec((1, 1), lambda i: (0, 0), memory_space=pltpu.SMEM),
            pl.BlockSpec((1, 1), lambda i: (0, 0), memory_space=pltpu.SMEM),
        ],
        out_specs=pl.BlockSpec((1, 1, _BLK), lambda i: (i, 0, 0)),
        out_shape=jax.ShapeDtypeStruct((nblk, 1, _BLK), jnp.float32),
        scratch_shapes=[pltpu.SMEM((1, 1), jnp.float32)],
    )(scalar_representation, W1, b1.reshape(d_hid, 1),
      W2.reshape(1, d_hid), b2.reshape(1, 1), n_arr)

    n_w = _SC_WORKERS
    spw = (-(-n_mol // n_w) + _LANES - 1) // _LANES * _LANES
    return _sc_csr_diff(n_mol, spw)(seg_m.astype(jnp.int32), p.reshape(npad))
